# baseline (speedup 1.0000x reference)
import functools

import jax
import jax.numpy as jnp
from jax import lax
from jax.experimental import pallas as pl
from jax.experimental.pallas import tpu as pltpu
from jax.experimental.pallas import tpu_sc as plsc

_NC = 2
_NS = 16
_NW = _NC * _NS
_CH = 80
_BLK = 8
_NBUF = 8
_LA = 4


def _gather_call(n_rows, v, d):
    n_per_w = n_rows // _NW
    n_chunks = n_per_w // _CH
    n_blocks = n_chunks // _BLK

    mesh = plsc.VectorSubcoreMesh(core_axis_name="c", subcore_axis_name="s")

    @functools.partial(
        pl.kernel,
        mesh=mesh,
        out_type=jax.ShapeDtypeStruct((n_rows, d), jnp.float32),
        scratch_types=[
            pltpu.VMEM((2, _BLK, _CH), jnp.int32),
            pltpu.VMEM((_NBUF, _CH, d), jnp.float32),
            pltpu.VMEM_SHARED((v, d), jnp.float32),
            pltpu.SemaphoreType.DMA,
            pltpu.SemaphoreType.DMA,
            pltpu.SemaphoreType.DMA,
        ],
    )
    def k(idx_hbm, table_hbm, out_hbm, idx_v, rows_v, table_spm,
          isem, gsem, wsem):
        wid = lax.axis_index("s") * _NC + lax.axis_index("c")
        chunk0 = wid * n_chunks

        @pl.when(lax.axis_index("s") == 0)
        def _stage_table():
            pltpu.sync_copy(table_hbm, table_spm)

        plsc.subcore_barrier()

        def idx_copy(blk, buf):
            return pltpu.async_copy(
                idx_hbm.at[pl.ds(chunk0 + blk * _BLK, _BLK)], idx_v.at[buf],
                isem)

        def wait_one_idx():
            pltpu.make_async_copy(
                idx_hbm.at[pl.ds(0, _BLK)], idx_v.at[0], isem).wait()

        def fire_gather(pb, j, b):
            pltpu.async_copy(
                table_spm.at[idx_v.at[pb].at[j]], rows_v.at[b], gsem)

        def wait_one_gather():
            pltpu.make_async_copy(
                out_hbm.at[pl.ds(0, _CH)], rows_v.at[0], gsem).wait()

        def fire_write(c_glb, b):
            pltpu.async_copy(
                rows_v.at[b], out_hbm.at[pl.ds(c_glb * _CH, _CH)], wsem)

        def wait_one_write():
            pltpu.make_async_copy(
                rows_v.at[0], out_hbm.at[pl.ds(0, _CH)], wsem).wait()

        def step(base, pb, j, first_block=False, last_block=False):
            wait_one_gather()
            fire_write(base + j, j)
            if not (first_block and j < _NBUF - _LA):
                wait_one_write()
            if j < _BLK - _LA:
                fire_gather(pb, j + _LA, (j + _LA) % _NBUF)
            elif not last_block:
                if j == _BLK - _LA:
                    wait_one_idx()
                fire_gather(1 - pb, j - (_BLK - _LA), (j + _LA) % _NBUF)
            elif j == _BLK - _LA:
                wait_one_idx()

        idx_copy(0, 0).wait()
        idx_copy(1, 1)
        for j in range(_LA):
            fire_gather(0, j, j)
        for j in range(_BLK):
            step(chunk0, 0, j, first_block=True)
        idx_copy(2, 0)

        def emit_block(blk, pb):
            base = chunk0 + blk * _BLK
            for j in range(_BLK):
                step(base, pb, j)
            idx_copy(jnp.minimum(blk + 2, n_blocks - 1), pb)

        def pair_body(q, carry):
            emit_block(1 + 2 * q, 1)
            emit_block(2 + 2 * q, 0)
            return carry

        lax.fori_loop(0, (n_blocks - 2) // 2, pair_body, 0)

        base = chunk0 + (n_blocks - 1) * _BLK
        for j in range(_BLK):
            step(base, (n_blocks - 1) % 2, j, last_block=True)
        for _ in range(_NBUF - _LA):
            wait_one_write()

    return k


def kernel(timesteps, table):
    b, s = timesteps.shape
    v, d = table.shape
    n = b * s
    idx2d = timesteps.reshape(n // _CH, _CH).astype(jnp.int32)
    out = _gather_call(n, v, d)(idx2d, table)
    return out.reshape(b, s, d)
